# baseline (speedup 1.0000x reference)
import functools

import jax
import jax.numpy as jnp
from jax import lax
from jax.experimental import pallas as pl
from jax.experimental.pallas import tpu as pltpu
from jax.experimental.pallas import tpu_sc as plsc

VOCAB = 1000000
DIM = 32
RANK = 8
SCALE = 1.0

NUM_CORES = 2
NUM_SUBCORES = 16
NW = NUM_CORES * NUM_SUBCORES
N_TOK = 16384 * 50
TOK_PER_W = N_TOK // NW
C = 1024
N_CHUNK = TOK_PER_W // C


def _body(weight_hbm, idx_hbm, lora_a_hbm, lora_b_hbm, out_hbm,
          idx_v, w_v, a_v, lb_v, sem_w, sem_a):
    wid = lax.axis_index("s") * NUM_CORES + lax.axis_index("c")
    base = wid * TOK_PER_W

    pltpu.sync_copy(lora_b_hbm, lb_v)

    lane = lax.iota(jnp.int32, 16)
    row_step = lax.select(lane >= 8, jnp.ones((16,), jnp.int32),
                          jnp.zeros((16,), jnp.int32))
    col_idx = lax.rem(lane, jnp.full((16,), RANK, jnp.int32))

    def chunk_body(ci, _):
        tok = pl.multiple_of(base + ci * C, C)
        pltpu.sync_copy(idx_hbm.at[pl.ds(tok, C)], idx_v)
        cp_w = pltpu.async_copy(weight_hbm.at[idx_v], w_v, sem_w)
        cp_a = pltpu.async_copy(lora_a_hbm.at[idx_v], a_v, sem_a)
        cp_w.wait()
        cp_a.wait()

        def pair_body(p, _):
            t0 = p * 2
            av = plsc.load_gather(a_v, [t0 + row_step, col_idx])
            acc00 = w_v[t0, pl.ds(0, 16)]
            acc01 = w_v[t0, pl.ds(16, 16)]
            acc10 = w_v[t0 + 1, pl.ds(0, 16)]
            acc11 = w_v[t0 + 1, pl.ds(16, 16)]
            for r in range(RANK):
                b0 = lb_v[r, pl.ds(0, 16)]
                b1 = lb_v[r, pl.ds(16, 16)]
                s0 = av[r] * SCALE
                s1 = av[r + RANK] * SCALE
                acc00 = acc00 + s0 * b0
                acc01 = acc01 + s0 * b1
                acc10 = acc10 + s1 * b0
                acc11 = acc11 + s1 * b1
            w_v[t0, pl.ds(0, 16)] = acc00
            w_v[t0, pl.ds(16, 16)] = acc01
            w_v[t0 + 1, pl.ds(0, 16)] = acc10
            w_v[t0 + 1, pl.ds(16, 16)] = acc11
            return 0

        lax.fori_loop(0, C // 2, pair_body, 0)
        pltpu.sync_copy(w_v, out_hbm.at[pl.ds(tok, C)])
        return 0

    lax.fori_loop(0, N_CHUNK, chunk_body, 0)


@jax.jit
def _lora_embed(weight, idx, lora_a, lora_b):
    mesh = plsc.VectorSubcoreMesh(core_axis_name="c", subcore_axis_name="s")
    fn = pl.kernel(
        _body,
        out_type=jax.ShapeDtypeStruct((N_TOK, DIM), jnp.float32),
        mesh=mesh,
        compiler_params=pltpu.CompilerParams(
            needs_layout_passes=False, use_tc_tiling_on_sc=False),
        scratch_types=[
            pltpu.VMEM((C,), jnp.int32),
            pltpu.VMEM((C, DIM), jnp.float32),
            pltpu.VMEM((C, RANK), jnp.float32),
            pltpu.VMEM((RANK, DIM), jnp.float32),
            pltpu.SemaphoreType.DMA,
            pltpu.SemaphoreType.DMA,
        ],
    )
    return fn(weight, idx, lora_a, lora_b)


def kernel(x, weight, lora_A, lora_B):
    idx = x.reshape(-1).astype(jnp.int32)
    out = _lora_embed(weight, idx, lora_A, lora_B)
    return out.reshape(x.shape[0], x.shape[1], DIM)


# speedup vs baseline: 1.5119x; 1.5119x over previous
import jax
import jax.numpy as jnp
from jax import lax
from jax.experimental import pallas as pl
from jax.experimental.pallas import tpu as pltpu
from jax.experimental.pallas import tpu_sc as plsc

VOCAB = 1000000
DIM = 32
RANK = 8
SCALE = 1.0

NUM_CORES = 2
NUM_SUBCORES = 16
NW = NUM_CORES * NUM_SUBCORES
N_TOK = 16384 * 50
TOK_PER_W = N_TOK // NW
CB = 16
C = CB * 50
N_CHUNK = TOK_PER_W // C
ROWS_PER_W = 16384 // NW


def _body(weight_hbm, idx_hbm, lora_a_hbm, lora_b_hbm, out_hbm,
          idx_v0, idx_v1, w_v0, w_v1, a_v0, a_v1, lb_v,
          sem_w0, sem_w1, sem_a0, sem_a1, sem_o0, sem_o1):
    wid = lax.axis_index("s") * NUM_CORES + lax.axis_index("c")
    base = wid * TOK_PER_W
    rbase = wid * ROWS_PER_W

    pltpu.sync_copy(lora_b_hbm, lb_v)

    lane = lax.iota(jnp.int32, 16)
    row_step = lax.select(lane >= 8, jnp.ones((16,), jnp.int32),
                          jnp.zeros((16,), jnp.int32))
    col_idx = lax.rem(lane, jnp.full((16,), RANK, jnp.int32))

    bufs = ((idx_v0, w_v0, a_v0, sem_w0, sem_a0, sem_o0),
            (idx_v1, w_v1, a_v1, sem_w1, sem_a1, sem_o1))

    def start_gathers(ci, b):
        idx_v, w_v, a_v, sem_w, sem_a, _ = bufs[b]
        tok = pl.multiple_of(base + ci * C, C)
        pltpu.sync_copy(idx_hbm.at[pl.ds(tok, C)], idx_v)
        pltpu.make_async_copy(weight_hbm.at[idx_v], w_v, sem_w).start()
        pltpu.make_async_copy(lora_a_hbm.at[idx_v], a_v, sem_a).start()

    def wait_gathers(b):
        idx_v, w_v, a_v, sem_w, sem_a, _ = bufs[b]
        pltpu.make_async_copy(weight_hbm.at[idx_v], w_v, sem_w).wait()
        pltpu.make_async_copy(lora_a_hbm.at[idx_v], a_v, sem_a).wait()

    def start_write(ci, b):
        _, w_v, _, _, _, sem_o = bufs[b]
        row = pl.multiple_of(rbase + ci * CB, CB)
        for j in range(CB):
            pltpu.make_async_copy(
                w_v.at[pl.ds(j * 50, 50)], out_hbm.at[row + j], sem_o).start()

    def wait_write(b):
        _, w_v, _, _, _, sem_o = bufs[b]
        for j in range(CB):
            pltpu.make_async_copy(
                w_v.at[pl.ds(0, 50)], out_hbm.at[0], sem_o).wait()

    def compute(b):
        _, w_v, a_v, _, _, _ = bufs[b]

        def pair_body(p):
            t0 = p * 2
            av = plsc.load_gather(a_v, [t0 + row_step, col_idx])
            acc00 = w_v[t0, pl.ds(0, 16)]
            acc01 = w_v[t0, pl.ds(16, 16)]
            acc10 = w_v[t0 + 1, pl.ds(0, 16)]
            acc11 = w_v[t0 + 1, pl.ds(16, 16)]
            for r in range(RANK):
                b0 = lb_v[r, pl.ds(0, 16)]
                b1 = lb_v[r, pl.ds(16, 16)]
                s0 = av[r] * SCALE
                s1 = av[r + RANK] * SCALE
                acc00 = acc00 + s0 * b0
                acc01 = acc01 + s0 * b1
                acc10 = acc10 + s1 * b0
                acc11 = acc11 + s1 * b1
            w_v[t0, pl.ds(0, 16)] = acc00
            w_v[t0, pl.ds(16, 16)] = acc01
            w_v[t0 + 1, pl.ds(0, 16)] = acc10
            w_v[t0 + 1, pl.ds(16, 16)] = acc11

        plsc.parallel_loop(0, C // 2, 1, unroll=4, carry=None)(pair_body)

    start_gathers(0, 0)

    def outer(cj, _):
        for b in (0, 1):
            ci = cj * 2 + b
            wait_gathers(b)
            nb = 1 - b
            @pl.when(ci >= 1)
            def _():
                wait_write(nb)

            @pl.when(ci + 1 < N_CHUNK)
            def _():
                start_gathers(ci + 1, nb)

            compute(b)
            start_write(ci, b)
        return 0

    lax.fori_loop(0, N_CHUNK // 2, outer, 0)
    wait_write(1)


@jax.jit
def _lora_embed(weight, idx, lora_a, lora_b):
    mesh = plsc.VectorSubcoreMesh(core_axis_name="c", subcore_axis_name="s")
    fn = pl.kernel(
        _body,
        out_type=jax.ShapeDtypeStruct((16384, 50, DIM), jnp.float32),
        mesh=mesh,
        compiler_params=pltpu.CompilerParams(
            needs_layout_passes=False, use_tc_tiling_on_sc=False),
        scratch_types=[
            pltpu.VMEM((C,), jnp.int32),
            pltpu.VMEM((C,), jnp.int32),
            pltpu.VMEM((C, DIM), jnp.float32),
            pltpu.VMEM((C, DIM), jnp.float32),
            pltpu.VMEM((C, RANK), jnp.float32),
            pltpu.VMEM((C, RANK), jnp.float32),
            pltpu.VMEM((RANK, DIM), jnp.float32),
            pltpu.SemaphoreType.DMA,
            pltpu.SemaphoreType.DMA,
            pltpu.SemaphoreType.DMA,
            pltpu.SemaphoreType.DMA,
            pltpu.SemaphoreType.DMA,
            pltpu.SemaphoreType.DMA,
        ],
    )
    return fn(weight, idx, lora_a, lora_b)


def kernel(x, weight, lora_A, lora_B):
    idx = x.reshape(-1).astype(jnp.int32)
    return _lora_embed(weight, idx, lora_A, lora_B)
